# baseline (speedup 1.0000x reference)
import functools

import jax
import jax.numpy as jnp
from jax import lax
from jax.experimental import pallas as pl
from jax.experimental.pallas import tpu as pltpu
from jax.experimental.pallas import tpu_sc as plsc

BATCH = 16384
N_FIELDS = 26
NC = 2
NS = 16
L = 16
NW = NC * NS
B_PER_W = BATCH // NW
IDX_PER_W = B_PER_W * N_FIELDS
GROUPS = B_PER_W // L

_mesh = plsc.VectorSubcoreMesh(
    core_axis_name="c", subcore_axis_name="s", num_cores=NC, num_subcores=NS
)


@functools.partial(
    pl.kernel,
    out_type=jax.ShapeDtypeStruct((BATCH,), jnp.float32),
    mesh=_mesh,
    scratch_types=[
        pltpu.VMEM((IDX_PER_W,), jnp.int32),
        pltpu.VMEM((IDX_PER_W,), jnp.float32),
        pltpu.VMEM((B_PER_W,), jnp.float32),
        pltpu.VMEM((L,), jnp.float32),
        pltpu.SemaphoreType.DMA,
    ],
    compiler_params=pltpu.CompilerParams(
        needs_layout_passes=False, use_tc_tiling_on_sc=True
    ),
)
def _lr_kernel(idx_hbm, w_hbm, b_hbm, out_hbm, idx_v, vals_v, out_v, b_v, sem):
    wid = lax.axis_index("s") * NC + lax.axis_index("c")
    base = wid * B_PER_W
    pltpu.sync_copy(b_hbm, b_v)
    idx_copies = [
        pltpu.async_copy(
            idx_hbm.at[f, pl.ds(base, B_PER_W)],
            idx_v.at[pl.ds(f * B_PER_W, B_PER_W)],
            sem,
        )
        for f in range(N_FIELDS)
    ]
    for c in idx_copies:
        c.wait()
    pltpu.async_copy(w_hbm.at[0].at[idx_v], vals_v, sem).wait()

    bvec = b_v[...]

    for g in range(GROUPS):
        accs = [bvec, 0.0, 0.0]
        for f in range(N_FIELDS):
            accs[f % 3] = accs[f % 3] + vals_v[pl.ds(f * B_PER_W + g * L, L)]
        acc = (accs[0] + accs[1]) + accs[2]
        y = 1.0 / (1.0 + jnp.exp(-acc))
        out_v[pl.ds(g * L, L)] = y

    pltpu.sync_copy(out_v, out_hbm.at[pl.ds(base, B_PER_W)])


def kernel(indices, w, b):
    idx_t = indices.T.astype(jnp.int32)
    w_t = w.T.astype(jnp.float32)
    b16 = jnp.broadcast_to(b.astype(jnp.float32), (L,))
    return _lr_kernel(idx_t, w_t, b16)


# speedup vs baseline: 1.0693x; 1.0693x over previous
import functools

import jax
import jax.numpy as jnp
from jax import lax
from jax.experimental import pallas as pl
from jax.experimental.pallas import tpu as pltpu
from jax.experimental.pallas import tpu_sc as plsc

BATCH = 16384
N_FIELDS = 26
NC = 2
NS = 16
L = 16
NW = NC * NS
B_PER_W = BATCH // NW
IDX_PER_W = B_PER_W * N_FIELDS
GROUPS = B_PER_W // L

_mesh = plsc.VectorSubcoreMesh(
    core_axis_name="c", subcore_axis_name="s", num_cores=NC, num_subcores=NS
)


@functools.partial(
    pl.kernel,
    out_type=jax.ShapeDtypeStruct((BATCH,), jnp.float32),
    mesh=_mesh,
    scratch_types=[
        pltpu.VMEM((IDX_PER_W,), jnp.int32),
        pltpu.VMEM((IDX_PER_W,), jnp.float32),
        pltpu.VMEM((B_PER_W,), jnp.float32),
        pltpu.VMEM((L,), jnp.float32),
        [pltpu.SemaphoreType.DMA for _ in range(2)],
        [pltpu.SemaphoreType.DMA for _ in range(2)],
    ],
    compiler_params=pltpu.CompilerParams(
        needs_layout_passes=False, use_tc_tiling_on_sc=True
    ),
)
def _lr_kernel(
    idx_hbm, w_hbm, b_hbm, out_hbm, idx_v, vals_v, out_v, b_v, isems, gsems
):
    wid = lax.axis_index("s") * NC + lax.axis_index("c")
    base = wid * B_PER_W
    HB = B_PER_W // 2
    HI = IDX_PER_W // 2
    pltpu.sync_copy(b_hbm, b_v)

    def fire_idx(h):
        return [
            pltpu.async_copy(
                idx_hbm.at[f, pl.ds(base + h * HB, HB)],
                idx_v.at[pl.ds(h * HI + f * HB, HB)],
                isems[h],
            )
            for f in range(N_FIELDS)
        ]

    def fire_gather(h):
        return pltpu.async_copy(
            w_hbm.at[0].at[idx_v.at[pl.ds(h * HI, HI)]],
            vals_v.at[pl.ds(h * HI, HI)],
            gsems[h],
        )

    idx0 = fire_idx(0)
    idx1 = fire_idx(1)
    for c in idx0:
        c.wait()
    g0 = fire_gather(0)
    for c in idx1:
        c.wait()
    g1 = fire_gather(1)

    bvec = b_v[...]

    def make_body(h):
        def body(g, carry):
            accs = [bvec, 0.0, 0.0]
            for f in range(N_FIELDS):
                accs[f % 3] = accs[f % 3] + vals_v[
                    pl.ds(h * HI + f * HB + g * L, L)
                ]
            acc = (accs[0] + accs[1]) + accs[2]
            y = 1.0 / (1.0 + jnp.exp(-acc))
            out_v[pl.ds(h * HB + g * L, L)] = y
            return carry

        return body

    g0.wait()
    lax.fori_loop(0, GROUPS // 2, make_body(0), 0)
    g1.wait()
    lax.fori_loop(0, GROUPS // 2, make_body(1), 0)
    pltpu.sync_copy(out_v, out_hbm.at[pl.ds(base, B_PER_W)])


def kernel(indices, w, b):
    idx_t = indices.T.astype(jnp.int32)
    w_t = w.T.astype(jnp.float32)
    b16 = jnp.broadcast_to(b.astype(jnp.float32), (L,))
    return _lr_kernel(idx_t, w_t, b16)
